# Initial kernel scaffold; baseline (speedup 1.0000x reference)
import jax
import jax.numpy as jnp
from jax.experimental import pallas as pl


def kernel(u, v, w, n, data_p, data_n, data_p_1, data_p_2, data_n_1, data_n_2, device, E_pos, E_neg, E_item, E_item_n):
    raise NotImplementedError("write your pallas kernel here")


# speedup vs baseline: 4.4239x; 4.4239x over previous
import functools

import jax
import jax.numpy as jnp
from jax import lax
from jax.experimental import pallas as pl
from jax.experimental.pallas import tpu as pltpu
from jax.experimental.pallas import tpu_sc as plsc

_M = 30000
_N = 20000
NN = _M + _N
DIM = 64
E = 800000
B = 8192

NC = 2
NS = 16
NW = NC * NS
L = 16

HALF = NN // 2
TRASH = HALF
ACC_ROWS = HALF + 16

EPT = 25024
E_PAD = NW * EPT
PCHUNK = 6256
NPCH = EPT // PCHUNK
CAP = 25600
CHUNK = 256
KSTR = CHUNK // 128

RPT = 1664
RCH = RPT // 128
LAST_START = HALF - RPT

_SENTINEL = 1 << 28

_mesh = plsc.VectorSubcoreMesh(core_axis_name="c", subcore_axis_name="s")

_i32 = jnp.int32
_f32 = jnp.float32


@functools.partial(
    pl.kernel,
    out_type=(
        jax.ShapeDtypeStruct((2, NW, CAP), _i32),
        jax.ShapeDtypeStruct((2, NW, CAP), _i32),
        jax.ShapeDtypeStruct((NW * 16,), _i32),
    ),
    mesh=_mesh,
    compiler_params=pltpu.CompilerParams(needs_layout_passes=False, use_tc_tiling_on_sc=False, internal_scratch_in_bytes=0),
    scratch_types=[
        pltpu.VMEM((PCHUNK,), _i32),
        pltpu.VMEM((PCHUNK,), _i32),
        pltpu.VMEM((CAP,), _i32),
        pltpu.VMEM((CAP,), _i32),
        pltpu.VMEM((CAP,), _i32),
        pltpu.VMEM((CAP,), _i32),
        pltpu.VMEM((16,), _i32),
    ],
)
def _partition(esrc, edst, sp, dp, counts, stage0, stage1,
               buf0s, buf0d, buf1s, buf1d, cntv):
    c = lax.axis_index("c")
    s = lax.axis_index("s")
    wid = c * NS + s
    base = wid * EPT

    def chunk_body(k, carry):
        pltpu.sync_copy(esrc.at[pl.ds(base + k * PCHUNK, PCHUNK)], stage0)
        pltpu.sync_copy(edst.at[pl.ds(base + k * PCHUNK, PCHUNK)], stage1)

        def vbody(j, cc):
            c0, c1 = cc
            sv = stage0[pl.ds(j * L, L)]
            dv = stage1[pl.ds(j * L, L)]
            m0 = dv < HALF
            m1 = jnp.logical_and(dv >= HALF, dv < NN)
            cs0 = plsc.cumsum(m0.astype(_i32))
            cs1 = plsc.cumsum(m1.astype(_i32))
            i0 = c0 + cs0 - 1
            i1 = c1 + cs1 - 1
            plsc.store_scatter(buf0s, [i0], sv, mask=m0)
            plsc.store_scatter(buf0d, [i0], dv, mask=m0)
            plsc.store_scatter(buf1s, [i1], sv, mask=m1)
            plsc.store_scatter(buf1d, [i1], dv - HALF, mask=m1)
            c0 = c0 + cs0[L - 1]
            c1 = c1 + cs1[L - 1]
            return (c0, c1)

        return lax.fori_loop(0, PCHUNK // L, vbody, carry)

    zero = jnp.zeros((), _i32)
    cnt0, cnt1 = lax.fori_loop(0, NPCH, chunk_body, (zero, zero))

    lanes = lax.iota(_i32, L)
    pads = wid * L + lanes
    padd = jnp.full((L,), TRASH, _i32)
    for t in range(CHUNK // L):
        buf0s[pl.ds(cnt0 + t * L, L)] = pads
        buf0d[pl.ds(cnt0 + t * L, L)] = padd
        buf1s[pl.ds(cnt1 + t * L, L)] = pads
        buf1d[pl.ds(cnt1 + t * L, L)] = padd
    up0 = jnp.left_shift(jnp.right_shift(cnt0 + CHUNK - 1, 8), 8)
    up1 = jnp.left_shift(jnp.right_shift(cnt1 + CHUNK - 1, 8), 8)
    cv = jnp.where(lanes == 0, up0, jnp.where(lanes == 1, up1, 0))
    cntv[...] = cv
    pltpu.sync_copy(cntv, counts.at[pl.ds(wid * 16, 16)])
    pltpu.sync_copy(buf0s, sp.at[0, wid])
    pltpu.sync_copy(buf0d, dp.at[0, wid])
    pltpu.sync_copy(buf1s, sp.at[1, wid])
    pltpu.sync_copy(buf1d, dp.at[1, wid])


@functools.cache
def _make_conv():
  return functools.partial(
    pl.kernel,
    out_type=jax.ShapeDtypeStruct((NN, DIM), _f32),
    mesh=_mesh,
    compiler_params=pltpu.CompilerParams(needs_layout_passes=False, use_tc_tiling_on_sc=False, internal_scratch_in_bytes=0),
    scratch_types=[
        pltpu.VMEM_SHARED((ACC_ROWS, DIM), _f32),
        pltpu.VMEM((NW * 16,), _i32),
        pltpu.VMEM((128,), _i32),
        pltpu.VMEM((128,), _i32),
        pltpu.VMEM((128,), _i32),
        pltpu.VMEM((128,), _i32),
        pltpu.VMEM((128, DIM), _f32),
        pltpu.VMEM((128, DIM), _f32),
        pltpu.SemaphoreType.DMA,
        pltpu.SemaphoreType.DMA,
    ],
  )(_conv_body)


def _conv(sp, dp, counts, x, idrows):
    return _make_conv()(sp, dp, counts, x, idrows)


def _conv_body(sp, dp, counts, x, idrows, y, acc, cntv, ix0, ix1, dx0, dx1,
               r0, r1, gsem, ssem):
    c = lax.axis_index("c")
    s = lax.axis_index("s")
    start_l = jnp.minimum(s * RPT, LAST_START)

    pltpu.sync_copy(counts, cntv)

    def init_body(k2, _):
        base = start_l + k2 * 128
        pltpu.sync_copy(idrows.at[pl.ds(base, 128)], dx0)
        pltpu.sync_copy(idrows.at[pl.ds(base + c * HALF, 128)], ix0)
        pltpu.async_copy(x.at[ix0], r0, gsem).wait()
        pltpu.async_copy(r0, acc.at[dx0], ssem).wait()
        return 0

    lax.fori_loop(0, RCH, init_body, 0)

    plsc.subcore_barrier()

    for rr in range(2):
        r = s * 2 + rr
        crow = cntv[pl.ds(r * L, L)]
        trips = jnp.right_shift(jnp.where(c == 0, crow[0], crow[1]), 8)

        def trip_body(t, _):
            e0 = t * CHUNK
            pltpu.sync_copy(sp.at[c, r, pl.ds(e0, 128)], ix0)
            pltpu.sync_copy(sp.at[c, r, pl.ds(e0 + 128, 128)], ix1)
            pltpu.sync_copy(dp.at[c, r, pl.ds(e0, 128)], dx0)
            pltpu.sync_copy(dp.at[c, r, pl.ds(e0 + 128, 128)], dx1)
            g0 = pltpu.async_copy(x.at[ix0], r0, gsem)
            g1 = pltpu.async_copy(x.at[ix1], r1, gsem)
            g0.wait()
            g1.wait()
            s0 = pltpu.async_copy(r0, acc.at[dx0], ssem, add=True)
            s1 = pltpu.async_copy(r1, acc.at[dx1], ssem, add=True)
            s0.wait()
            s1.wait()
            return 0

        lax.fori_loop(0, trips, trip_body, 0)

    plsc.subcore_barrier()

    def out_body(k2, _):
        base = start_l + k2 * 128
        pltpu.sync_copy(idrows.at[pl.ds(base, 128)], dx0)
        pltpu.sync_copy(idrows.at[pl.ds(base + c * HALF, 128)], ix0)
        pltpu.async_copy(acc.at[dx0], r0, gsem).wait()
        pltpu.async_copy(r0, y.at[ix0], ssem).wait()
        return 0

    lax.fori_loop(0, RCH, out_body, 0)


_SCORE_CH = 128
_N_SCH = B // (NW * _SCORE_CH)


@functools.partial(
    pl.kernel,
    out_type=jax.ShapeDtypeStruct((B,), _f32),
    mesh=_mesh,
    compiler_params=pltpu.CompilerParams(needs_layout_passes=False, use_tc_tiling_on_sc=False, internal_scratch_in_bytes=0),
    scratch_types=[
        pltpu.VMEM((3, _SCORE_CH), _i32),
        pltpu.VMEM((_SCORE_CH,), _f32),
        pltpu.VMEM((3, _SCORE_CH, DIM), _f32),
        pltpu.VMEM((6, _SCORE_CH, DIM), _f32),
        pltpu.VMEM((_SCORE_CH, L), _f32),
        pltpu.VMEM((_SCORE_CH, L), _f32),
        pltpu.VMEM((_SCORE_CH + L,), _f32),
        pltpu.VMEM((_SCORE_CH + L,), _f32),
        pltpu.VMEM((_SCORE_CH,), _f32),
        pltpu.SemaphoreType.DMA,
    ],
)
def _score(u, v, n, w, b0p, b0n, x1_0, x1_1, x1_2, x1_3, x1_4, x1_5,
           x2_0, x2_1, x2_2, x2_3, x2_4, x2_5, res,
           idx, wbuf, b0r, xr, posv, negv, pospad, negpad, outb, sem):
    c = lax.axis_index("c")
    s = lax.axis_index("s")
    wid = c * NS + s
    b0s = (b0p, b0n)
    x1s = (x1_0, x1_1, x1_2, x1_3, x1_4, x1_5)
    x2s = (x2_0, x2_1, x2_2, x2_3, x2_4, x2_5)
    side_graphs = ((0, 2, 3), (1, 4, 5))

    for ch in range(_N_SCH):
        boff = wid * (_N_SCH * _SCORE_CH) + ch * _SCORE_CH
        pltpu.sync_copy(u.at[pl.ds(boff, _SCORE_CH)], idx.at[0])
        pltpu.sync_copy(v.at[pl.ds(boff, _SCORE_CH)], idx.at[1])
        pltpu.sync_copy(n.at[pl.ds(boff, _SCORE_CH)], idx.at[2])
        pltpu.sync_copy(w.at[pl.ds(boff, _SCORE_CH)], wbuf)

        first = True
        for side in range(2):
            b0t = b0s[side]
            gds = [
                pltpu.async_copy(b0t.at[idx.at[q]], b0r.at[q], sem)
                for q in range(3)
            ]
            for d in gds:
                d.wait()
            for g in side_graphs[side]:
                gds = [
                    pltpu.async_copy(x1s[g].at[idx.at[q]], xr.at[q], sem)
                    for q in range(3)
                ] + [
                    pltpu.async_copy(x2s[g].at[idx.at[q]], xr.at[3 + q], sem)
                    for q in range(3)
                ]
                for d in gds:
                    d.wait()

                accumulate = not first
                first = False

                def bbody(b, _, accumulate=accumulate):
                    pacc = jnp.zeros((L,), _f32)
                    nacc = jnp.zeros((L,), _f32)
                    for q in range(DIM // L):
                        sl = pl.ds(q * L, L)
                        zu = b0r[0, b, sl] + xr[0, b, sl] + xr[3, b, sl]
                        zvv = b0r[1, b, sl] + xr[1, b, sl] + xr[4, b, sl]
                        zn = b0r[2, b, sl] + xr[2, b, sl] + xr[5, b, sl]
                        pacc = pacc + zu * zvv
                        nacc = nacc + zu * zn
                    if accumulate:
                        pacc = pacc + posv[b, pl.ds(0, L)]
                        nacc = nacc + negv[b, pl.ds(0, L)]
                    posv[b, pl.ds(0, L)] = pacc
                    negv[b, pl.ds(0, L)] = nacc
                    return 0

                lax.fori_loop(0, _SCORE_CH, bbody, 0)

        lanes = lax.iota(_i32, L)
        m_last = lanes == (L - 1)

        def rbody(b, _):
            bidx = jnp.full((L,), b, _i32)
            cs = plsc.cumsum(posv[b, pl.ds(0, L)])
            plsc.store_scatter(pospad, [bidx], cs, mask=m_last)
            cs = plsc.cumsum(negv[b, pl.ds(0, L)])
            plsc.store_scatter(negpad, [bidx], cs, mask=m_last)
            return 0

        lax.fori_loop(0, _SCORE_CH, rbody, 0)

        for q in range(_SCORE_CH // L):
            sl = pl.ds(q * L, L)
            outb[sl] = (wbuf[sl] * pospad[sl] - negpad[sl]) * _f32(1.0 / 9.0)
        pltpu.sync_copy(outb, res.at[pl.ds(boff, _SCORE_CH)])


def _sc_layout(a):
    return a


def kernel(u, v, w, n, data_p, data_n, data_p_1, data_p_2, data_n_1, data_n_2,
           device, E_pos, E_neg, E_item, E_item_n):
    del device
    b0p = _sc_layout(jnp.concatenate([E_pos, E_item], axis=0))
    b0n = _sc_layout(jnp.concatenate([E_neg, E_item_n], axis=0))
    idrows = jnp.arange(NN, dtype=_i32)
    fill_src = jnp.zeros((E_PAD - E,), _i32)
    fill_dst = jnp.full((E_PAD - E,), _SENTINEL, _i32)
    graphs = (data_p, data_n, data_p_1, data_p_2, data_n_1, data_n_2)
    bases = (b0p, b0n, b0p, b0p, b0n, b0n)
    x1s, x2s = [], []
    for ei, b0 in zip(graphs, bases):
        esrc = jnp.concatenate([ei[0], fill_src])
        edst = jnp.concatenate([ei[1], fill_dst])
        sp, dp, cnts = _partition(esrc, edst)
        x1 = _sc_layout(_conv(sp, dp, cnts, b0, idrows))
        x2 = _sc_layout(_conv(sp, dp, cnts, x1, idrows))
        x1s.append(x1)
        x2s.append(x2)
    return _score(u, v, n, w, b0p, b0n, *x1s, *x2s)


# speedup vs baseline: 7.6466x; 1.7285x over previous
import functools

import jax
import jax.numpy as jnp
from jax import lax
from jax.experimental import pallas as pl
from jax.experimental.pallas import tpu as pltpu
from jax.experimental.pallas import tpu_sc as plsc

_M = 30000
_N = 20000
NN = _M + _N
DIM = 64
E = 800000
B = 8192

NC = 2
NS = 16
NW = NC * NS
L = 16

HALF = NN // 2
TRASH = HALF
ACC_ROWS = HALF + 16

EPT = 25024
E_PAD = NW * EPT
PCHUNK = 6256
NPCH = EPT // PCHUNK
CAP = 25600
CHUNK = 128

RPT = 1664
RCH = RPT // 128
LAST_START = HALF - RPT

_SENTINEL = 1 << 28

_mesh = plsc.VectorSubcoreMesh(core_axis_name="c", subcore_axis_name="s")

_i32 = jnp.int32
_f32 = jnp.float32


@functools.partial(
    pl.kernel,
    out_type=(
        jax.ShapeDtypeStruct((2, NW, CAP), _i32),
        jax.ShapeDtypeStruct((2, NW, CAP), _i32),
        jax.ShapeDtypeStruct((NW * 16,), _i32),
    ),
    mesh=_mesh,
    compiler_params=pltpu.CompilerParams(needs_layout_passes=False, use_tc_tiling_on_sc=False, internal_scratch_in_bytes=0),
    scratch_types=[
        pltpu.VMEM((PCHUNK,), _i32),
        pltpu.VMEM((PCHUNK,), _i32),
        pltpu.VMEM((CAP,), _i32),
        pltpu.VMEM((CAP,), _i32),
        pltpu.VMEM((CAP,), _i32),
        pltpu.VMEM((CAP,), _i32),
        pltpu.VMEM((16,), _i32),
    ],
)
def _partition(esrc, edst, sp, dp, counts, stage0, stage1,
               buf0s, buf0d, buf1s, buf1d, cntv):
    c = lax.axis_index("c")
    s = lax.axis_index("s")
    wid = c * NS + s
    base = wid * EPT

    def chunk_body(k, carry):
        pltpu.sync_copy(esrc.at[pl.ds(base + k * PCHUNK, PCHUNK)], stage0)
        pltpu.sync_copy(edst.at[pl.ds(base + k * PCHUNK, PCHUNK)], stage1)

        def vbody(j, cc):
            c0, c1 = cc
            sv = stage0[pl.ds(j * L, L)]
            dv = stage1[pl.ds(j * L, L)]
            m0 = dv < HALF
            m1 = jnp.logical_and(dv >= HALF, dv < NN)
            cs0 = plsc.cumsum(m0.astype(_i32))
            cs1 = plsc.cumsum(m1.astype(_i32))
            i0 = c0 + cs0 - 1
            i1 = c1 + cs1 - 1
            plsc.store_scatter(buf0s, [i0], sv, mask=m0)
            plsc.store_scatter(buf0d, [i0], dv, mask=m0)
            plsc.store_scatter(buf1s, [i1], sv, mask=m1)
            plsc.store_scatter(buf1d, [i1], dv - HALF, mask=m1)
            c0 = c0 + cs0[L - 1]
            c1 = c1 + cs1[L - 1]
            return (c0, c1)

        return lax.fori_loop(0, PCHUNK // L, vbody, carry)

    zero = jnp.zeros((), _i32)
    cnt0, cnt1 = lax.fori_loop(0, NPCH, chunk_body, (zero, zero))

    lanes = lax.iota(_i32, L)
    pads = wid * L + lanes
    padd = jnp.full((L,), TRASH, _i32)
    for t in range(CHUNK // L):
        buf0s[pl.ds(cnt0 + t * L, L)] = pads
        buf0d[pl.ds(cnt0 + t * L, L)] = padd
        buf1s[pl.ds(cnt1 + t * L, L)] = pads
        buf1d[pl.ds(cnt1 + t * L, L)] = padd
    up0 = jnp.left_shift(jnp.right_shift(cnt0 + 127, 7), 7)
    up1 = jnp.left_shift(jnp.right_shift(cnt1 + 127, 7), 7)
    cv = jnp.where(lanes == 0, up0, jnp.where(lanes == 1, up1, 0))
    cntv[...] = cv
    pltpu.sync_copy(cntv, counts.at[pl.ds(wid * 16, 16)])
    pltpu.sync_copy(buf0s, sp.at[0, wid])
    pltpu.sync_copy(buf0d, dp.at[0, wid])
    pltpu.sync_copy(buf1s, sp.at[1, wid])
    pltpu.sync_copy(buf1d, dp.at[1, wid])


@functools.cache
def _make_conv():
  return functools.partial(
    pl.kernel,
    out_type=jax.ShapeDtypeStruct((NN, DIM), _f32),
    mesh=_mesh,
    compiler_params=pltpu.CompilerParams(needs_layout_passes=False, use_tc_tiling_on_sc=False, internal_scratch_in_bytes=0),
    scratch_types=[
        pltpu.VMEM_SHARED((ACC_ROWS, DIM), _f32),
        pltpu.VMEM((NW * 16,), _i32),
    ]
    + [pltpu.VMEM((128,), _i32)] * 6
    + [pltpu.VMEM((128, DIM), _f32)] * 3
    + [pltpu.SemaphoreType.DMA] * 9,
  )(_conv_body)


def _conv(sp, dp, counts, x, idrows):
    return _make_conv()(sp, dp, counts, x, idrows)


def _conv_body(sp, dp, counts, x, idrows, y, acc, cntv,
               ixa, ixb, ixc, dxa, dxb, dxc, ra, rb, rc,
               ia, ib, ic, ga, gb, gc, sa, sb, sc_):
    c = lax.axis_index("c")
    s = lax.axis_index("s")
    ix = (ixa, ixb, ixc)
    dx = (dxa, dxb, dxc)
    rbuf = (ra, rb, rc)
    isem = (ia, ib, ic)
    gsem = (ga, gb, gc)
    ssem = (sa, sb, sc_)
    start_l = jnp.minimum(s * RPT, LAST_START)

    pltpu.sync_copy(counts, cntv)

    def init_body(k2, _):
        base = start_l + k2 * 128
        pltpu.sync_copy(idrows.at[pl.ds(base, 128)], dxa)
        pltpu.sync_copy(idrows.at[pl.ds(base + c * HALF, 128)], ixa)
        pltpu.async_copy(x.at[ixa], ra, ga).wait()
        pltpu.async_copy(ra, acc.at[dxa], sa).wait()
        return 0

    lax.fori_loop(0, RCH, init_body, 0)

    plsc.subcore_barrier()

    for rr in range(2):
        r = s * 2 + rr
        crow = cntv[pl.ds(r * L, L)]
        trips = jnp.right_shift(jnp.where(c == 0, crow[0], crow[1]), 7)
        ngroups = (trips + 2) // 3

        def group(tt, _):
            valid = [tt * 3 + k < trips for k in range(3)]
            for k in range(3):
                @pl.when(jnp.logical_and(valid[k], tt > 0))
                def _(k=k):
                    pltpu.make_async_copy(rbuf[k], acc.at[dx[k]],
                                          ssem[k]).wait()

                @pl.when(valid[k])
                def _(k=k):
                    e0 = (tt * 3 + k) * 128
                    pltpu.async_copy(sp.at[c, r, pl.ds(e0, 128)], ix[k],
                                     isem[k])
                    pltpu.async_copy(dp.at[c, r, pl.ds(e0, 128)], dx[k],
                                     isem[k])
            for k in range(3):
                @pl.when(valid[k])
                def _(k=k):
                    e0 = (tt * 3 + k) * 128
                    pltpu.make_async_copy(sp.at[c, r, pl.ds(e0, 128)],
                                          ix[k], isem[k]).wait()
                    pltpu.make_async_copy(dp.at[c, r, pl.ds(e0, 128)],
                                          dx[k], isem[k]).wait()
                    pltpu.async_copy(x.at[ix[k]], rbuf[k], gsem[k])
            for k in range(3):
                @pl.when(valid[k])
                def _(k=k):
                    pltpu.make_async_copy(x.at[ix[k]], rbuf[k],
                                          gsem[k]).wait()
                    pltpu.async_copy(rbuf[k], acc.at[dx[k]], ssem[k],
                                     add=True)
            return 0

        lax.fori_loop(0, ngroups, group, 0)
        for k in range(3):
            @pl.when(jnp.logical_and(ngroups > 0,
                                     (ngroups - 1) * 3 + k < trips))
            def _(k=k):
                pltpu.make_async_copy(rbuf[k], acc.at[dx[k]], ssem[k]).wait()

    plsc.subcore_barrier()

    def out_body(k2, _):
        base = start_l + k2 * 128
        pltpu.sync_copy(idrows.at[pl.ds(base, 128)], dxa)
        pltpu.sync_copy(idrows.at[pl.ds(base + c * HALF, 128)], ixa)
        pltpu.async_copy(acc.at[dxa], ra, ga).wait()
        pltpu.async_copy(ra, y.at[ixa], sa).wait()
        return 0

    lax.fori_loop(0, RCH, out_body, 0)


_SCORE_CH = 128
_N_SCH = B // (NW * _SCORE_CH)


@functools.partial(
    pl.kernel,
    out_type=jax.ShapeDtypeStruct((B,), _f32),
    mesh=_mesh,
    compiler_params=pltpu.CompilerParams(needs_layout_passes=False, use_tc_tiling_on_sc=False, internal_scratch_in_bytes=0),
    scratch_types=[
        pltpu.VMEM((3, _SCORE_CH), _i32),
        pltpu.VMEM((_SCORE_CH,), _f32),
        pltpu.VMEM((3, _SCORE_CH, DIM), _f32),
        pltpu.VMEM((6, _SCORE_CH, DIM), _f32),
        pltpu.VMEM((_SCORE_CH, L), _f32),
        pltpu.VMEM((_SCORE_CH, L), _f32),
        pltpu.VMEM((_SCORE_CH + L,), _f32),
        pltpu.VMEM((_SCORE_CH + L,), _f32),
        pltpu.VMEM((_SCORE_CH,), _f32),
        pltpu.SemaphoreType.DMA,
    ],
)
def _score(u, v, n, w, b0p, b0n, x1_0, x1_1, x1_2, x1_3, x1_4, x1_5,
           x2_0, x2_1, x2_2, x2_3, x2_4, x2_5, res,
           idx, wbuf, b0r, xr, posv, negv, pospad, negpad, outb, sem):
    c = lax.axis_index("c")
    s = lax.axis_index("s")
    wid = c * NS + s
    b0s = (b0p, b0n)
    x1s = (x1_0, x1_1, x1_2, x1_3, x1_4, x1_5)
    x2s = (x2_0, x2_1, x2_2, x2_3, x2_4, x2_5)
    side_graphs = ((0, 2, 3), (1, 4, 5))

    for ch in range(_N_SCH):
        boff = wid * (_N_SCH * _SCORE_CH) + ch * _SCORE_CH
        pltpu.sync_copy(u.at[pl.ds(boff, _SCORE_CH)], idx.at[0])
        pltpu.sync_copy(v.at[pl.ds(boff, _SCORE_CH)], idx.at[1])
        pltpu.sync_copy(n.at[pl.ds(boff, _SCORE_CH)], idx.at[2])
        pltpu.sync_copy(w.at[pl.ds(boff, _SCORE_CH)], wbuf)

        first = True
        for side in range(2):
            b0t = b0s[side]
            gds = [
                pltpu.async_copy(b0t.at[idx.at[q]], b0r.at[q], sem)
                for q in range(3)
            ]
            for d in gds:
                d.wait()
            for g in side_graphs[side]:
                gds = [
                    pltpu.async_copy(x1s[g].at[idx.at[q]], xr.at[q], sem)
                    for q in range(3)
                ] + [
                    pltpu.async_copy(x2s[g].at[idx.at[q]], xr.at[3 + q], sem)
                    for q in range(3)
                ]
                for d in gds:
                    d.wait()

                accumulate = not first
                first = False

                def bbody(b, _, accumulate=accumulate):
                    pacc = jnp.zeros((L,), _f32)
                    nacc = jnp.zeros((L,), _f32)
                    for q in range(DIM // L):
                        sl = pl.ds(q * L, L)
                        zu = b0r[0, b, sl] + xr[0, b, sl] + xr[3, b, sl]
                        zvv = b0r[1, b, sl] + xr[1, b, sl] + xr[4, b, sl]
                        zn = b0r[2, b, sl] + xr[2, b, sl] + xr[5, b, sl]
                        pacc = pacc + zu * zvv
                        nacc = nacc + zu * zn
                    if accumulate:
                        pacc = pacc + posv[b, pl.ds(0, L)]
                        nacc = nacc + negv[b, pl.ds(0, L)]
                    posv[b, pl.ds(0, L)] = pacc
                    negv[b, pl.ds(0, L)] = nacc
                    return 0

                lax.fori_loop(0, _SCORE_CH, bbody, 0)

        lanes = lax.iota(_i32, L)
        m_last = lanes == (L - 1)

        def rbody(b, _):
            bidx = jnp.full((L,), b, _i32)
            cs = plsc.cumsum(posv[b, pl.ds(0, L)])
            plsc.store_scatter(pospad, [bidx], cs, mask=m_last)
            cs = plsc.cumsum(negv[b, pl.ds(0, L)])
            plsc.store_scatter(negpad, [bidx], cs, mask=m_last)
            return 0

        lax.fori_loop(0, _SCORE_CH, rbody, 0)

        for q in range(_SCORE_CH // L):
            sl = pl.ds(q * L, L)
            outb[sl] = (wbuf[sl] * pospad[sl] - negpad[sl]) * _f32(1.0 / 9.0)
        pltpu.sync_copy(outb, res.at[pl.ds(boff, _SCORE_CH)])


def _sc_layout(a):
    return a


def kernel(u, v, w, n, data_p, data_n, data_p_1, data_p_2, data_n_1, data_n_2,
           device, E_pos, E_neg, E_item, E_item_n):
    del device
    b0p = _sc_layout(jnp.concatenate([E_pos, E_item], axis=0))
    b0n = _sc_layout(jnp.concatenate([E_neg, E_item_n], axis=0))
    idrows = jnp.arange(NN, dtype=_i32)
    fill_src = jnp.zeros((E_PAD - E,), _i32)
    fill_dst = jnp.full((E_PAD - E,), _SENTINEL, _i32)
    graphs = (data_p, data_n, data_p_1, data_p_2, data_n_1, data_n_2)
    bases = (b0p, b0n, b0p, b0p, b0n, b0n)
    x1s, x2s = [], []
    for ei, b0 in zip(graphs, bases):
        esrc = jnp.concatenate([ei[0], fill_src])
        edst = jnp.concatenate([ei[1], fill_dst])
        sp, dp, cnts = _partition(esrc, edst)
        x1 = _sc_layout(_conv(sp, dp, cnts, b0, idrows))
        x2 = _sc_layout(_conv(sp, dp, cnts, x1, idrows))
        x1s.append(x1)
        x2s.append(x2)
    return _score(u, v, n, w, b0p, b0n, *x1s, *x2s)


# speedup vs baseline: 8.9218x; 1.1668x over previous
import functools

import jax
import jax.numpy as jnp
from jax import lax
from jax.experimental import pallas as pl
from jax.experimental.pallas import tpu as pltpu
from jax.experimental.pallas import tpu_sc as plsc

_M = 30000
_N = 20000
NN = _M + _N
DIM = 64
E = 800000
B = 8192

NC = 2
NS = 16
NW = NC * NS
L = 16

HALF = NN // 2
TRASH = HALF
ACC_ROWS = HALF + 16

EPT = 25024
E_PAD = NW * EPT
PCHUNK = 6256
NPCH = EPT // PCHUNK
CAP = 25600
CHUNK = 128

RPT = 1664
RCH = RPT // 128
LAST_START = HALF - RPT

_SENTINEL = 1 << 28

_mesh = plsc.VectorSubcoreMesh(core_axis_name="c", subcore_axis_name="s")

_i32 = jnp.int32
_f32 = jnp.float32


@functools.partial(
    pl.kernel,
    out_type=(
        jax.ShapeDtypeStruct((2, NW, CAP), _i32),
        jax.ShapeDtypeStruct((2, NW, CAP), _i32),
        jax.ShapeDtypeStruct((NW * 16,), _i32),
    ),
    mesh=_mesh,
    compiler_params=pltpu.CompilerParams(needs_layout_passes=False, use_tc_tiling_on_sc=False, internal_scratch_in_bytes=0),
    scratch_types=[
        pltpu.VMEM((PCHUNK,), _i32),
        pltpu.VMEM((PCHUNK,), _i32),
        pltpu.VMEM((CAP,), _i32),
        pltpu.VMEM((CAP,), _i32),
        pltpu.VMEM((CAP,), _i32),
        pltpu.VMEM((CAP,), _i32),
        pltpu.VMEM((16,), _i32),
    ],
)
def _partition(esrc, edst, sp, dp, counts, stage0, stage1,
               buf0s, buf0d, buf1s, buf1d, cntv):
    c = lax.axis_index("c")
    s = lax.axis_index("s")
    wid = c * NS + s
    base = wid * EPT

    def chunk_body(k, carry):
        pltpu.sync_copy(esrc.at[pl.ds(base + k * PCHUNK, PCHUNK)], stage0)
        pltpu.sync_copy(edst.at[pl.ds(base + k * PCHUNK, PCHUNK)], stage1)

        def vbody(j, cc):
            c0, c1 = cc
            sv = stage0[pl.ds(j * L, L)]
            dv = stage1[pl.ds(j * L, L)]
            m0 = dv < HALF
            m1 = jnp.logical_and(dv >= HALF, dv < NN)
            cs0 = plsc.cumsum(m0.astype(_i32))
            cs1 = plsc.cumsum(m1.astype(_i32))
            i0 = c0 + cs0 - 1
            i1 = c1 + cs1 - 1
            plsc.store_scatter(buf0s, [i0], sv, mask=m0)
            plsc.store_scatter(buf0d, [i0], dv, mask=m0)
            plsc.store_scatter(buf1s, [i1], sv, mask=m1)
            plsc.store_scatter(buf1d, [i1], dv - HALF, mask=m1)
            c0 = c0 + cs0[L - 1]
            c1 = c1 + cs1[L - 1]
            return (c0, c1)

        return lax.fori_loop(0, PCHUNK // L, vbody, carry)

    zero = jnp.zeros((), _i32)
    cnt0, cnt1 = lax.fori_loop(0, NPCH, chunk_body, (zero, zero))

    lanes = lax.iota(_i32, L)
    pads = wid * L + lanes
    padd = jnp.full((L,), TRASH, _i32)
    for t in range(CHUNK // L):
        buf0s[pl.ds(cnt0 + t * L, L)] = pads
        buf0d[pl.ds(cnt0 + t * L, L)] = padd
        buf1s[pl.ds(cnt1 + t * L, L)] = pads
        buf1d[pl.ds(cnt1 + t * L, L)] = padd
    up0 = jnp.left_shift(jnp.right_shift(cnt0 + 127, 7), 7)
    up1 = jnp.left_shift(jnp.right_shift(cnt1 + 127, 7), 7)
    cv = jnp.where(lanes == 0, up0, jnp.where(lanes == 1, up1, 0))
    cntv[...] = cv
    pltpu.sync_copy(cntv, counts.at[pl.ds(wid * 16, 16)])
    pltpu.sync_copy(buf0s, sp.at[0, wid])
    pltpu.sync_copy(buf0d, dp.at[0, wid])
    pltpu.sync_copy(buf1s, sp.at[1, wid])
    pltpu.sync_copy(buf1d, dp.at[1, wid])


@functools.cache
def _make_conv():
  return functools.partial(
    pl.kernel,
    out_type=jax.ShapeDtypeStruct((NN, DIM), _f32),
    mesh=_mesh,
    compiler_params=pltpu.CompilerParams(needs_layout_passes=False, use_tc_tiling_on_sc=False, internal_scratch_in_bytes=0),
    scratch_types=[
        pltpu.VMEM_SHARED((ACC_ROWS, DIM), _f32),
        pltpu.VMEM((NW * 16,), _i32),
    ]
    + [pltpu.VMEM((128,), _i32)] * 6
    + [pltpu.VMEM((128, DIM), _f32)] * 3
    + [pltpu.SemaphoreType.DMA] * 9,
  )(_conv_body)


def _conv(sp, dp, counts, x, idrows):
    return _make_conv()(sp, dp, counts, x, idrows)


def _conv_body(sp, dp, counts, x, idrows, y, acc, cntv,
               ixa, ixb, ixc, dxa, dxb, dxc, ra, rb, rc,
               ia, ib, ic, ga, gb, gc, sa, sb, sc_):
    c = lax.axis_index("c")
    s = lax.axis_index("s")
    ix = (ixa, ixb, ixc)
    dx = (dxa, dxb, dxc)
    rbuf = (ra, rb, rc)
    isem = (ia, ib, ic)
    gsem = (ga, gb, gc)
    ssem = (sa, sb, sc_)
    start_l = jnp.minimum(s * RPT, LAST_START)

    pltpu.sync_copy(counts, cntv)

    def _ident_pipe(src_ref, dst_ref, src_glob):
        for gidx in range((RCH + 2) // 3):
            for k in range(3):
                cidx = gidx * 3 + k
                if cidx >= RCH:
                    continue
                if gidx > 0:
                    pltpu.make_async_copy(rbuf[k], dst_ref.at[dx[k]],
                                          ssem[k]).wait()
                base = start_l + cidx * 128
                gb_ = base + c * HALF if src_glob else base
                sb_ = base if src_glob else base + c * HALF
                pltpu.async_copy(idrows.at[pl.ds(gb_, 128)], ix[k], isem[k])
                pltpu.async_copy(idrows.at[pl.ds(sb_, 128)], dx[k], isem[k])
            for k in range(3):
                cidx = gidx * 3 + k
                if cidx >= RCH:
                    continue
                base = start_l + cidx * 128
                gb_ = base + c * HALF if src_glob else base
                sb_ = base if src_glob else base + c * HALF
                pltpu.make_async_copy(idrows.at[pl.ds(gb_, 128)], ix[k],
                                      isem[k]).wait()
                pltpu.make_async_copy(idrows.at[pl.ds(sb_, 128)], dx[k],
                                      isem[k]).wait()
                pltpu.async_copy(src_ref.at[ix[k]], rbuf[k], gsem[k])
            for k in range(3):
                cidx = gidx * 3 + k
                if cidx >= RCH:
                    continue
                pltpu.make_async_copy(src_ref.at[ix[k]], rbuf[k],
                                      gsem[k]).wait()
                pltpu.async_copy(rbuf[k], dst_ref.at[dx[k]], ssem[k])
        for k in range(min(3, RCH)):
            pltpu.make_async_copy(rbuf[k], dst_ref.at[dx[k]],
                                  ssem[k]).wait()

    _ident_pipe(x, acc, True)

    plsc.subcore_barrier()

    for rr in range(2):
        r = s * 2 + rr
        crow = cntv[pl.ds(r * L, L)]
        trips = jnp.right_shift(jnp.where(c == 0, crow[0], crow[1]), 7)
        ngroups = (trips + 2) // 3

        def group(tt, _):
            valid = [tt * 3 + k < trips for k in range(3)]
            for k in range(3):
                @pl.when(jnp.logical_and(valid[k], tt > 0))
                def _(k=k):
                    pltpu.make_async_copy(rbuf[k], acc.at[dx[k]],
                                          ssem[k]).wait()

                @pl.when(valid[k])
                def _(k=k):
                    e0 = (tt * 3 + k) * 128
                    pltpu.async_copy(sp.at[c, r, pl.ds(e0, 128)], ix[k],
                                     isem[k])
                    pltpu.async_copy(dp.at[c, r, pl.ds(e0, 128)], dx[k],
                                     isem[k])
            for k in range(3):
                @pl.when(valid[k])
                def _(k=k):
                    e0 = (tt * 3 + k) * 128
                    pltpu.make_async_copy(sp.at[c, r, pl.ds(e0, 128)],
                                          ix[k], isem[k]).wait()
                    pltpu.make_async_copy(dp.at[c, r, pl.ds(e0, 128)],
                                          dx[k], isem[k]).wait()
                    pltpu.async_copy(x.at[ix[k]], rbuf[k], gsem[k])
            for k in range(3):
                @pl.when(valid[k])
                def _(k=k):
                    pltpu.make_async_copy(x.at[ix[k]], rbuf[k],
                                          gsem[k]).wait()
                    pltpu.async_copy(rbuf[k], acc.at[dx[k]], ssem[k],
                                     add=True)
            return 0

        lax.fori_loop(0, ngroups, group, 0)
        for k in range(3):
            @pl.when(trips > k)
            def _(k=k):
                pltpu.make_async_copy(rbuf[k], acc.at[dx[k]], ssem[k]).wait()

    plsc.subcore_barrier()

    _ident_pipe(acc, y, False)


_SCORE_CH = 128
_N_SCH = B // (NW * _SCORE_CH)


@functools.partial(
    pl.kernel,
    out_type=jax.ShapeDtypeStruct((B,), _f32),
    mesh=_mesh,
    compiler_params=pltpu.CompilerParams(needs_layout_passes=False, use_tc_tiling_on_sc=False, internal_scratch_in_bytes=0),
    scratch_types=[
        pltpu.VMEM((3, _SCORE_CH), _i32),
        pltpu.VMEM((_SCORE_CH,), _f32),
        pltpu.VMEM((3, _SCORE_CH, DIM), _f32),
        pltpu.VMEM((6, _SCORE_CH, DIM), _f32),
        pltpu.VMEM((_SCORE_CH, L), _f32),
        pltpu.VMEM((_SCORE_CH, L), _f32),
        pltpu.VMEM((_SCORE_CH + L,), _f32),
        pltpu.VMEM((_SCORE_CH + L,), _f32),
        pltpu.VMEM((_SCORE_CH,), _f32),
        pltpu.SemaphoreType.DMA,
    ],
)
def _score(u, v, n, w, b0p, b0n, x1_0, x1_1, x1_2, x1_3, x1_4, x1_5,
           x2_0, x2_1, x2_2, x2_3, x2_4, x2_5, res,
           idx, wbuf, b0r, xr, posv, negv, pospad, negpad, outb, sem):
    c = lax.axis_index("c")
    s = lax.axis_index("s")
    wid = c * NS + s
    b0s = (b0p, b0n)
    x1s = (x1_0, x1_1, x1_2, x1_3, x1_4, x1_5)
    x2s = (x2_0, x2_1, x2_2, x2_3, x2_4, x2_5)
    side_graphs = ((0, 2, 3), (1, 4, 5))

    for ch in range(_N_SCH):
        boff = wid * (_N_SCH * _SCORE_CH) + ch * _SCORE_CH
        pltpu.sync_copy(u.at[pl.ds(boff, _SCORE_CH)], idx.at[0])
        pltpu.sync_copy(v.at[pl.ds(boff, _SCORE_CH)], idx.at[1])
        pltpu.sync_copy(n.at[pl.ds(boff, _SCORE_CH)], idx.at[2])
        pltpu.sync_copy(w.at[pl.ds(boff, _SCORE_CH)], wbuf)

        first = True
        for side in range(2):
            b0t = b0s[side]
            gds = [
                pltpu.async_copy(b0t.at[idx.at[q]], b0r.at[q], sem)
                for q in range(3)
            ]
            for d in gds:
                d.wait()
            for g in side_graphs[side]:
                gds = [
                    pltpu.async_copy(x1s[g].at[idx.at[q]], xr.at[q], sem)
                    for q in range(3)
                ] + [
                    pltpu.async_copy(x2s[g].at[idx.at[q]], xr.at[3 + q], sem)
                    for q in range(3)
                ]
                for d in gds:
                    d.wait()

                accumulate = not first
                first = False

                def bbody(b, _, accumulate=accumulate):
                    pacc = jnp.zeros((L,), _f32)
                    nacc = jnp.zeros((L,), _f32)
                    for q in range(DIM // L):
                        sl = pl.ds(q * L, L)
                        zu = b0r[0, b, sl] + xr[0, b, sl] + xr[3, b, sl]
                        zvv = b0r[1, b, sl] + xr[1, b, sl] + xr[4, b, sl]
                        zn = b0r[2, b, sl] + xr[2, b, sl] + xr[5, b, sl]
                        pacc = pacc + zu * zvv
                        nacc = nacc + zu * zn
                    if accumulate:
                        pacc = pacc + posv[b, pl.ds(0, L)]
                        nacc = nacc + negv[b, pl.ds(0, L)]
                    posv[b, pl.ds(0, L)] = pacc
                    negv[b, pl.ds(0, L)] = nacc
                    return 0

                lax.fori_loop(0, _SCORE_CH, bbody, 0)

        lanes = lax.iota(_i32, L)
        m_last = lanes == (L - 1)

        def rbody(b, _):
            bidx = jnp.full((L,), b, _i32)
            cs = plsc.cumsum(posv[b, pl.ds(0, L)])
            plsc.store_scatter(pospad, [bidx], cs, mask=m_last)
            cs = plsc.cumsum(negv[b, pl.ds(0, L)])
            plsc.store_scatter(negpad, [bidx], cs, mask=m_last)
            return 0

        lax.fori_loop(0, _SCORE_CH, rbody, 0)

        for q in range(_SCORE_CH // L):
            sl = pl.ds(q * L, L)
            outb[sl] = (wbuf[sl] * pospad[sl] - negpad[sl]) * _f32(1.0 / 9.0)
        pltpu.sync_copy(outb, res.at[pl.ds(boff, _SCORE_CH)])


def _sc_layout(a):
    return a


def kernel(u, v, w, n, data_p, data_n, data_p_1, data_p_2, data_n_1, data_n_2,
           device, E_pos, E_neg, E_item, E_item_n):
    del device
    b0p = _sc_layout(jnp.concatenate([E_pos, E_item], axis=0))
    b0n = _sc_layout(jnp.concatenate([E_neg, E_item_n], axis=0))
    idrows = jnp.arange(NN, dtype=_i32)
    fill_src = jnp.zeros((E_PAD - E,), _i32)
    fill_dst = jnp.full((E_PAD - E,), _SENTINEL, _i32)
    graphs = (data_p, data_n, data_p_1, data_p_2, data_n_1, data_n_2)
    bases = (b0p, b0n, b0p, b0p, b0n, b0n)
    x1s, x2s = [], []
    for ei, b0 in zip(graphs, bases):
        esrc = jnp.concatenate([ei[0], fill_src])
        edst = jnp.concatenate([ei[1], fill_dst])
        sp, dp, cnts = _partition(esrc, edst)
        x1 = _sc_layout(_conv(sp, dp, cnts, b0, idrows))
        x2 = _sc_layout(_conv(sp, dp, cnts, x1, idrows))
        x1s.append(x1)
        x2s.append(x2)
    return _score(u, v, n, w, b0p, b0n, *x1s, *x2s)
